# baseline (speedup 1.0000x reference)
import functools

import jax
import jax.numpy as jnp
from jax import lax
from jax.experimental import pallas as pl
from jax.experimental.pallas import tpu as pltpu
from jax.experimental.pallas import tpu_sc as plsc

_ALPHABET = 33
_OUT_CH = _ALPHABET + 1
_LANES = 16
_NUM_CORES = 2
_NUM_SUBCORES = 16
_NUM_WORKERS = _NUM_CORES * _NUM_SUBCORES
_TR = 8
_TC = 128
_NBUF = 3
_NTOK = 5


@functools.cache
def _build(b: int, t: int):
    rblocks = b // _TR
    w_per_row = max(_NUM_WORKERS // rblocks, 1)
    cb_per_w = (t // _TC) // w_per_row
    cpw = _TC // _LANES

    mesh = plsc.VectorSubcoreMesh(
        core_axis_name="c", subcore_axis_name="s",
        num_cores=_NUM_CORES, num_subcores=_NUM_SUBCORES)

    @functools.partial(
        pl.kernel,
        out_type=jax.ShapeDtypeStruct((_OUT_CH, b, t), jnp.float32),
        mesh=mesh,
        scratch_types=(
            [pltpu.VMEM((_TR, _TC), jnp.int32) for _ in range(_NTOK)]
            + [pltpu.VMEM((_TR, _TC), jnp.float32) for _ in range(2)]
            + [pltpu.VMEM((_OUT_CH, _TR, _TC), jnp.float32)
               for _ in range(_NBUF)]
            + [pltpu.SemaphoreType.DMA for _ in range(_NBUF + 2)]
        ),
        compiler_params=pltpu.CompilerParams(needs_layout_passes=False),
    )
    def sc_kernel(tok_hbm, eng_hbm, out_hbm,
                  tv0, tv1, tv2, tv3, tv4, ev0, ev1, ob0, ob1, ob2,
                  so0, so1, so2, si0, si1):
        wid = lax.axis_index("s") * _NUM_CORES + lax.axis_index("c")
        rb = wid // w_per_row
        cb0 = (wid % w_per_row) * cb_per_w
        r0 = rb * _TR
        iota = lax.iota(jnp.int32, _LANES)
        ones = jnp.full((_LANES,), 1.0, jnp.float32)
        zeros = jnp.zeros((_LANES,), jnp.float32)
        obufs = (ob0, ob1, ob2)
        osems = (so0, so1, so2)
        tbufs = (tv0, tv1, tv2, tv3, tv4)
        ebufs = (ev0, ev1)
        isems = (si0, si1)
        odescs = [None] * _NBUF
        idescs = [None, None]

        def start_inputs(k):
            c0 = (cb0 + k) * _TC
            d1 = pltpu.async_copy(
                tok_hbm.at[pl.ds(r0, _TR), pl.ds(c0, _TC)],
                tbufs[k % _NTOK], isems[k % 2])
            d2 = pltpu.async_copy(
                eng_hbm.at[pl.ds(r0, _TR), pl.ds(c0, _TC)],
                ebufs[k % 2], isems[k % 2])
            idescs[k % 2] = (d1, d2)

        start_inputs(0)
        for k in range(cb_per_w):
            ob = obufs[k % _NBUF]
            if odescs[k % _NBUF] is not None:
                odescs[k % _NBUF].wait()

            if k < _NBUF:
                def zero_body(ch, _, ob=ob):
                    for r in range(_TR):
                        for cblk in range(cpw):
                            ob[ch, r, pl.ds(cblk * _LANES, _LANES)] = zeros
                    return 0

                lax.fori_loop(0, _ALPHABET, zero_body, 0)

            for d in idescs[k % 2]:
                d.wait()
            tok_v = tbufs[k % _NTOK]
            eng_v = ebufs[k % 2]
            tok_p = tbufs[(k - _NBUF) % _NTOK]
            if k + 1 < cb_per_w:
                start_inputs(k + 1)

            for r in range(_TR):
                rvec = jnp.full((_LANES,), r, jnp.int32)

                def col_body(cblk, _, ob=ob, tok_v=tok_v, eng_v=eng_v,
                             tok_p=tok_p, r=r, rvec=rvec, undo=(k >= _NBUF)):
                    cstart = cblk * _LANES
                    cvec = cstart + iota
                    if undo:
                        tp = tok_p[r, pl.ds(cstart, _LANES)]
                        validp = jnp.logical_and(tp >= 0, tp < _ALPHABET)
                        plsc.store_scatter(ob, [tp, rvec, cvec], zeros,
                                           mask=validp)
                    tok = tok_v[r, pl.ds(cstart, _LANES)]
                    eng = eng_v[r, pl.ds(cstart, _LANES)]
                    valid = jnp.logical_and(tok >= 0, tok < _ALPHABET)
                    plsc.store_scatter(ob, [tok, rvec, cvec], ones,
                                       mask=valid)
                    motif = jnp.where(eng <= -1.0, jnp.float32(1.0),
                                      jnp.float32(0.0))
                    ob[_ALPHABET, r, pl.ds(cstart, _LANES)] = motif
                    return 0

                lax.fori_loop(0, cpw, col_body, 0)

            c0 = (cb0 + k) * _TC
            odescs[k % _NBUF] = pltpu.async_copy(
                ob, out_hbm.at[:, pl.ds(r0, _TR), pl.ds(c0, _TC)],
                osems[k % _NBUF])

        for d in odescs:
            if d is not None:
                d.wait()

    return sc_kernel


def kernel(tokens, energy_scores):
    b, t = tokens.shape
    out = _build(b, t)(tokens.astype(jnp.int32), energy_scores)
    return jnp.transpose(out, (1, 2, 0))
